# baseline (speedup 1.0000x reference)
import functools

import jax
import jax.numpy as jnp
from jax import lax
from jax.experimental import pallas as pl
from jax.experimental.pallas import tpu as pltpu
from jax.experimental.pallas import tpu_sc as plsc

_GATHER_DNUMS = lax.GatherDimensionNumbers(
    offset_dims=(), collapsed_slice_dims=(0,), start_index_map=(0,))


def _lane_perm(v, p):
    return lax.gather(v, p[:, None], _GATHER_DNUMS, slice_sizes=(1,),
                      mode=lax.GatherScatterMode.PROMISE_IN_BOUNDS)


def _sc_run(B, Bsc, D, NC, NS):
    NW = NC * NS
    rows_per_w = Bsc // NW
    C = 64
    n_chunks = rows_per_w // C
    n_seg = D // 16
    nbuf = 3
    unroll = 4

    mesh = plsc.VectorSubcoreMesh(core_axis_name="c", subcore_axis_name="s",
                                  num_cores=NC)

    @functools.partial(
        pl.kernel,
        mesh=mesh,
        out_type=(
            jax.ShapeDtypeStruct((Bsc, D), jnp.float32),
            jax.ShapeDtypeStruct((Bsc, D), jnp.float32),
        ),
        scratch_types=[
            pltpu.VMEM((rows_per_w,), jnp.int32),
            pltpu.VMEM((nbuf, C, D), jnp.float32),
            pltpu.VMEM((nbuf, C, D), jnp.float32),
        ] + [pltpu.SemaphoreType.DMA] * (4 * nbuf),
    )
    def run(x_hbm, rel_hbm, w_tab, out1, out2, idx_all, x_b, w_b, *sems):
        sw, sx = sems[0:nbuf], sems[nbuf:2 * nbuf]
        so1, so2 = sems[2 * nbuf:3 * nbuf], sems[3 * nbuf:4 * nbuf]
        wid = lax.axis_index("s") * NC + lax.axis_index("c")
        base = wid * rows_per_w
        pltpu.sync_copy(rel_hbm.at[pl.ds(base, rows_per_w)], idx_all)
        lanes = lax.iota(jnp.int32, 16)
        perms = [(lanes + sh) & 15 for sh in (8, 4, 2, 1)]

        in_cp = {}
        out_cp = {}

        def issue_in(c):
            b = c % nbuf
            off = base + c * C
            gw = pltpu.async_copy(
                w_tab.at[idx_all.at[pl.ds(c * C, C)]], w_b.at[b], sw[b])
            gx = pltpu.async_copy(x_hbm.at[pl.ds(off, C), :], x_b.at[b], sx[b])
            in_cp[c] = (gw, gx)

        for c in range(n_chunks):
            b = c % nbuf
            if c == 0:
                for k in range(min(nbuf - 1, n_chunks)):
                    issue_in(k)
            p = c + nbuf - 1
            if p < n_chunks:
                if p >= nbuf:
                    for cp in out_cp[p - nbuf]:
                        cp.wait()
                issue_in(p)
            gw, gx = in_cp.pop(c)
            gx.wait()
            gw.wait()
            off = base + c * C
            o2 = pltpu.async_copy(w_b.at[b], out2.at[pl.ds(off, C), :], so2[b])

            def rows(i, carry):
                for rr in range(unroll):
                    r = i * unroll + rr
                    xs = [x_b[b, r, pl.ds(16 * s, 16)] for s in range(n_seg)]
                    ws = [w_b[b, r, pl.ds(16 * s, 16)] for s in range(n_seg)]
                    acc = xs[0] * ws[0]
                    for s in range(1, n_seg):
                        acc = acc + xs[s] * ws[s]
                    for pm in perms:
                        acc = acc + _lane_perm(acc, pm)
                    for s in range(n_seg):
                        x_b[b, r, pl.ds(16 * s, 16)] = xs[s] - acc * ws[s]
                return carry

            lax.fori_loop(0, C // unroll, rows, 0)
            o1 = pltpu.async_copy(x_b.at[b], out1.at[pl.ds(off, C), :], so1[b])
            out_cp[c] = (o1, o2)

        for c in range(max(0, n_chunks - nbuf), n_chunks):
            for cp in out_cp[c]:
                cp.wait()

    return run


def _tc_run(B, Bsc, D, V, R=256):
    nblk = (B - Bsc) // R
    k0 = Bsc // R

    def body(rel_ref, x_ref, w_ref, out1_ref, out2_ref):
        rel = rel_ref[0]
        cols = lax.broadcasted_iota(jnp.int32, (R, V), 1)
        onehot = (cols == rel).astype(jnp.bfloat16)
        w_r = jnp.dot(onehot, w_ref[...], preferred_element_type=jnp.float32)
        xv = x_ref[...]
        dot = jnp.sum(w_r * xv, axis=1, keepdims=True)
        out1_ref[...] = xv - dot * w_r
        out2_ref[...] = w_r

    return pl.pallas_call(
        body,
        grid=(nblk,),
        in_specs=[
            pl.BlockSpec((1, R, 1), lambda i: (i + k0, 0, 0)),
            pl.BlockSpec((R, D), lambda i: (i + k0, 0)),
            pl.BlockSpec((V, D), lambda i: (0, 0)),
        ],
        out_specs=[
            pl.BlockSpec((R, D), lambda i: (i + k0, 0)),
            pl.BlockSpec((R, D), lambda i: (i + k0, 0)),
        ],
        out_shape=(
            jax.ShapeDtypeStruct((B, D), jnp.float32),
            jax.ShapeDtypeStruct((B, D), jnp.float32),
        ),
    )


def kernel(x, relation, W):
    B, D = x.shape
    V = 1024
    Btc = 8192
    Bsc = B - Btc
    rel32 = relation.astype(jnp.int32)
    info = plsc.get_sparse_core_info()

    sc_out1, sc_out2 = _sc_run(B, Bsc, D, info.num_cores, info.num_subcores)(
        x, rel32, W)

    w_pad = jnp.zeros((V, D), jnp.bfloat16).at[:W.shape[0]].set(
        W.astype(jnp.bfloat16))
    rel3 = rel32.reshape(B // 256, 256, 1)
    tc_out1, tc_out2 = _tc_run(B, Bsc, D, V)(rel3, x, w_pad)

    out1 = lax.dynamic_update_slice(tc_out1, sc_out1, (0, 0))
    out2 = lax.dynamic_update_slice(tc_out2, sc_out2, (0, 0))
    return (out1, out2)


# speedup vs baseline: 1.3663x; 1.3663x over previous
import functools

import jax
import jax.numpy as jnp
from jax import lax
from jax.experimental import pallas as pl
from jax.experimental.pallas import tpu as pltpu
from jax.experimental.pallas import tpu_sc as plsc

_GATHER_DNUMS = lax.GatherDimensionNumbers(
    offset_dims=(), collapsed_slice_dims=(0,), start_index_map=(0,))


def _lane_perm(v, p):
    return lax.gather(v, p[:, None], _GATHER_DNUMS, slice_sizes=(1,),
                      mode=lax.GatherScatterMode.PROMISE_IN_BOUNDS)


def _sc_run(B, Bsc, D, NC, NS):
    NW = NC * NS
    rows_per_w = Bsc // NW
    C = 64
    n_chunks = rows_per_w // C
    n_seg = D // 16
    nbuf = 3
    unroll = 4

    mesh = plsc.VectorSubcoreMesh(core_axis_name="c", subcore_axis_name="s",
                                  num_cores=NC)

    @functools.partial(
        pl.kernel,
        mesh=mesh,
        out_type=(
            jax.ShapeDtypeStruct((B, D), jnp.float32),
            jax.ShapeDtypeStruct((B, D), jnp.float32),
        ),
        scratch_types=[
            pltpu.VMEM((rows_per_w,), jnp.int32),
            pltpu.VMEM((nbuf, C, D), jnp.float32),
            pltpu.VMEM((nbuf, C, D), jnp.float32),
        ] + [pltpu.SemaphoreType.DMA] * (4 * nbuf),
    )
    def run(x_hbm, rel_hbm, w_tab, out1, out2, idx_all, x_b, w_b, *sems):
        sw, sx = sems[0:nbuf], sems[nbuf:2 * nbuf]
        so1, so2 = sems[2 * nbuf:3 * nbuf], sems[3 * nbuf:4 * nbuf]
        wid = lax.axis_index("s") * NC + lax.axis_index("c")
        base = wid * rows_per_w
        pltpu.sync_copy(rel_hbm.at[pl.ds(base, rows_per_w)], idx_all)
        lanes = lax.iota(jnp.int32, 16)
        perms = [(lanes + sh) & 15 for sh in (8, 4, 2, 1)]

        in_cp = {}
        out_cp = {}

        def issue_in(c):
            b = c % nbuf
            off = base + c * C
            gw = pltpu.async_copy(
                w_tab.at[idx_all.at[pl.ds(c * C, C)]], w_b.at[b], sw[b])
            gx = pltpu.async_copy(x_hbm.at[pl.ds(off, C), :], x_b.at[b], sx[b])
            in_cp[c] = (gw, gx)

        for c in range(n_chunks):
            b = c % nbuf
            if c == 0:
                for k in range(min(nbuf - 1, n_chunks)):
                    issue_in(k)
            p = c + nbuf - 1
            if p < n_chunks:
                if p >= nbuf:
                    for cp in out_cp[p - nbuf]:
                        cp.wait()
                issue_in(p)
            gw, gx = in_cp.pop(c)
            gx.wait()
            gw.wait()
            off = base + c * C
            o2 = pltpu.async_copy(w_b.at[b], out2.at[pl.ds(off, C), :], so2[b])

            def rows(i, carry):
                for rr in range(unroll):
                    r = i * unroll + rr
                    xs = [x_b[b, r, pl.ds(16 * s, 16)] for s in range(n_seg)]
                    ws = [w_b[b, r, pl.ds(16 * s, 16)] for s in range(n_seg)]
                    acc = xs[0] * ws[0]
                    for s in range(1, n_seg):
                        acc = acc + xs[s] * ws[s]
                    for pm in perms:
                        acc = acc + _lane_perm(acc, pm)
                    for s in range(n_seg):
                        x_b[b, r, pl.ds(16 * s, 16)] = xs[s] - acc * ws[s]
                return carry

            lax.fori_loop(0, C // unroll, rows, 0)
            o1 = pltpu.async_copy(x_b.at[b], out1.at[pl.ds(off, C), :], so1[b])
            out_cp[c] = (o1, o2)

        for c in range(max(0, n_chunks - nbuf), n_chunks):
            for cp in out_cp[c]:
                cp.wait()

    return run


def _tc_run(B, Bsc, D, V, R=256):
    nblk = (B - Bsc) // R
    k0 = Bsc // R

    def body(rel_ref, x_ref, w_ref, out1_ref, out2_ref):
        relr = rel_ref[0]
        rows = lax.broadcasted_iota(jnp.int32, (V, R), 0)
        onehot_t = (rows == relr).astype(jnp.bfloat16)
        w_r = lax.dot_general(
            onehot_t, w_ref[...], (((0,), (0,)), ((), ())),
            preferred_element_type=jnp.float32)
        xv = x_ref[...]
        dot = jnp.sum(w_r * xv, axis=1, keepdims=True)
        out1_ref[...] = xv - dot * w_r
        out2_ref[...] = w_r

    return pl.pallas_call(
        body,
        grid=(nblk,),
        in_specs=[
            pl.BlockSpec((1, 1, R), lambda i: (i + k0, 0, 0)),
            pl.BlockSpec((R, D), lambda i: (i + k0, 0)),
            pl.BlockSpec((V, D), lambda i: (0, 0)),
        ],
        out_specs=[
            pl.BlockSpec((R, D), lambda i: (i, 0)),
            pl.BlockSpec((R, D), lambda i: (i, 0)),
        ],
        out_shape=(
            jax.ShapeDtypeStruct((B - Bsc, D), jnp.float32),
            jax.ShapeDtypeStruct((B - Bsc, D), jnp.float32),
        ),
    )


def kernel(x, relation, W):
    B, D = x.shape
    V = 1024
    R = 256
    Btc = 6144
    Bsc = B - Btc
    rel32 = relation.astype(jnp.int32)
    info = plsc.get_sparse_core_info()

    sc_out1, sc_out2 = _sc_run(B, Bsc, D, info.num_cores, info.num_subcores)(
        x, rel32, W)

    w_pad = jnp.pad(W.astype(jnp.bfloat16), ((0, V - W.shape[0]), (0, 0)))
    rel3 = rel32.reshape(B // R, 1, R)
    tc_out1, tc_out2 = _tc_run(B, Bsc, D, V, R)(rel3, x, w_pad)

    out1 = lax.dynamic_update_slice(sc_out1, tc_out1, (Bsc, 0))
    out2 = lax.dynamic_update_slice(sc_out2, tc_out2, (Bsc, 0))
    return (out1, out2)
